# Initial kernel scaffold; baseline (speedup 1.0000x reference)
import jax
import jax.numpy as jnp
from jax.experimental import pallas as pl


def kernel(h_code, c_code, joint_w, joint_s, joint_b, res0_w1, res0_s1, res0_b1, res0_w2, res0_s2, res0_b2, res1_w1, res1_s1, res1_b1, res1_w2, res1_s2, res1_b2, up_w, up_s, up_b):
    raise NotImplementedError("write your pallas kernel here")


# speedup vs baseline: 1.4730x; 1.4730x over previous
import jax
import jax.numpy as jnp
import numpy as np
from jax.experimental import pallas as pl
from jax.experimental.pallas import tpu as pltpu


def _sigmoid(v):
    return 1.0 / (1.0 + jnp.exp(-v))


def _flat9(w):
    cout, cin = w.shape[:2]
    return jnp.transpose(w, (0, 2, 3, 1)).reshape(cout, 9 * cin)


def _make_body(ngf, nef, H, W):
    HW = H * W
    C = ngf
    C2 = 2 * ngf
    UC = 2 * (ngf // 2)
    UH = UC // 2

    def col_stack(x, m_first, m_last):
        cin = x.shape[0]
        z1 = jnp.zeros((cin, 1), x.dtype)
        left = jnp.concatenate([z1, x[:, :-1]], axis=1) * m_first
        right = jnp.concatenate([x[:, 1:], z1], axis=1) * m_last
        return jnp.concatenate([left, x, right], axis=0)

    def row_shift(x3, dr):
        zr = jnp.zeros((x3.shape[0], W), x3.dtype)
        if dr == -1:
            return jnp.concatenate([zr, x3[:, :-W]], axis=1)
        return jnp.concatenate([x3[:, W:], zr], axis=1)

    def stack9(x, m_first, m_last):
        x3 = col_stack(x, m_first, m_last)
        return jnp.concatenate(
            [row_shift(x3, -1), x3, row_shift(x3, 1)], axis=0)

    def body(ct_ref, h_ref, mask_ref, cw_ref, wj_ref, bj_ref,
             w01_ref, b01_ref, w02_ref, b02_ref,
             w11_ref, b11_ref, w12_ref, b12_ref,
             wu_ref, bu_ref, o_ref):
        masks = mask_ref[...]
        m_first, m_last = masks[0:1], masks[1:2]
        m_top, m_bot = masks[2:3], masks[3:4]

        x = h_ref[0]

        x9 = stack9(x, m_first, m_last)
        acc = jnp.dot(wj_ref[...], x9,
                      preferred_element_type=jnp.float32)

        vt = jnp.dot(cw_ref[...], ct_ref[0],
                     preferred_element_type=jnp.float32)
        corr = bj_ref[...]
        row_ms = (m_top, None, m_bot)
        col_ms = (m_first, None, m_last)
        for a in range(3):
            s = None
            for b in range(3):
                t = a * 3 + b
                v = vt[t * C2:(t + 1) * C2]
                term = v * col_ms[b] if col_ms[b] is not None else v
                s = term if s is None else s + term
            if row_ms[a] is not None:
                s = s * row_ms[a]
            corr = corr + s
        acc = acc + corr
        y = acc[:C] * _sigmoid(acc[C:])

        def res(yin, w1_ref, b1_ref, w2_ref, b2_ref):
            y9 = stack9(yin, m_first, m_last)
            a1 = jnp.dot(w1_ref[...], y9,
                         preferred_element_type=jnp.float32) + b1_ref[...]
            g = a1[:C] * _sigmoid(a1[C:])
            g9 = stack9(g, m_first, m_last)
            a2 = jnp.dot(w2_ref[...], g9,
                         preferred_element_type=jnp.float32) + b2_ref[...]
            return a2 + yin

        z = res(y, w01_ref, b01_ref, w02_ref, b02_ref)
        z = res(z, w11_ref, b11_ref, w12_ref, b12_ref)

        z3 = col_stack(z, m_first, m_last)
        zm = row_shift(z3, -1)
        zp = row_shift(z3, 1)
        for ph, (py, px) in enumerate(((0, 0), (0, 1), (1, 0), (1, 1))):
            za, zb = (zm, z3) if py == 0 else (z3, zp)
            s0 = 0 if px == 0 else C
            xin = jnp.concatenate(
                [za[s0:s0 + 2 * C], zb[s0:s0 + 2 * C]], axis=0)
            aph = jnp.dot(wu_ref[ph], xin,
                          preferred_element_type=jnp.float32) + bu_ref[...]
            o_ref[0, ph] = aph[:UH] * _sigmoid(aph[UH:])

    return body


def kernel(h_code, c_code, joint_w, joint_s, joint_b,
           res0_w1, res0_s1, res0_b1, res0_w2, res0_s2, res0_b2,
           res1_w1, res1_s1, res1_b1, res1_w2, res1_s2, res1_b2,
           up_w, up_s, up_b):
    N, ngf, s, _ = h_code.shape
    nef = c_code.shape[1]
    H = W = s
    HW = H * W
    f32 = jnp.float32
    C, C2 = ngf, 2 * ngf
    UC = 2 * (ngf // 2)

    x = h_code.astype(f32).reshape(N, C, HW)
    ct = c_code.astype(f32)[:, :, None]

    wj_f = joint_w.astype(f32) * joint_s.astype(f32)[:, None, None, None]
    wc = wj_f[:, :nef]
    wh = wj_f[:, nef:]
    cw = jnp.transpose(wc, (2, 3, 0, 1)).reshape(9 * C2, nef)
    wj = _flat9(wh)
    bj = joint_b.astype(f32).reshape(C2, 1)

    def res_prep(w1, s1, b1, w2, s2, b2):
        w1f = w1.astype(f32) * s1.astype(f32)[:, None, None, None]
        w2f = w2.astype(f32) * s2.astype(f32)[:, None, None, None]
        return (_flat9(w1f), b1.astype(f32).reshape(C2, 1),
                _flat9(w2f), b2.astype(f32).reshape(C, 1))

    w01, b01, w02, b02 = res_prep(res0_w1, res0_s1, res0_b1,
                                  res0_w2, res0_s2, res0_b2)
    w11, b11, w12, b12 = res_prep(res1_w1, res1_s1, res1_b1,
                                  res1_w2, res1_s2, res1_b2)

    wu_f = up_w.astype(f32) * up_s.astype(f32)[:, None, None, None]
    w9u = jnp.transpose(wu_f, (2, 3, 0, 1)).reshape(9, UC, C)
    row_sets = (((0,), (1, 2)), ((0, 1), (2,)))
    col_sets = (((0,), (1, 2)), ((0, 1), (2,)))
    phases = []
    for py in range(2):
        for px in range(2):
            blocks = []
            for a in range(2):
                for b in range(2):
                    acc = sum(w9u[kh * 3 + kw]
                              for kh in row_sets[py][a]
                              for kw in col_sets[px][b])
                    blocks.append(acc)
            phases.append(jnp.concatenate(blocks, axis=1))
    wu = jnp.stack(phases)
    bu = up_b.astype(f32).reshape(UC, 1)

    col = np.arange(HW) % W
    row = np.arange(HW) // W
    masks = jnp.asarray(np.stack([
        col != 0, col != W - 1, row != 0, row != H - 1]).astype(np.float32))

    body = _make_body(ngf, nef, H, W)
    UHalf = UC // 2

    out4 = pl.pallas_call(
        body,
        out_shape=jax.ShapeDtypeStruct((N, 4, UHalf, HW), f32),
        grid=(N,),
        in_specs=[
            pl.BlockSpec((1, nef, 1), lambda n: (n, 0, 0)),
            pl.BlockSpec((1, C, HW), lambda n: (n, 0, 0)),
            pl.BlockSpec((4, HW), lambda n: (0, 0)),
            pl.BlockSpec((9 * C2, nef), lambda n: (0, 0)),
            pl.BlockSpec((C2, 9 * C), lambda n: (0, 0)),
            pl.BlockSpec((C2, 1), lambda n: (0, 0)),
            pl.BlockSpec((C2, 9 * C), lambda n: (0, 0)),
            pl.BlockSpec((C2, 1), lambda n: (0, 0)),
            pl.BlockSpec((C, 9 * C), lambda n: (0, 0)),
            pl.BlockSpec((C, 1), lambda n: (0, 0)),
            pl.BlockSpec((C2, 9 * C), lambda n: (0, 0)),
            pl.BlockSpec((C2, 1), lambda n: (0, 0)),
            pl.BlockSpec((C, 9 * C), lambda n: (0, 0)),
            pl.BlockSpec((C, 1), lambda n: (0, 0)),
            pl.BlockSpec((4, UC, 4 * C), lambda n: (0, 0, 0)),
            pl.BlockSpec((UC, 1), lambda n: (0, 0)),
        ],
        out_specs=pl.BlockSpec((1, 4, UHalf, HW), lambda n: (n, 0, 0, 0)),
        compiler_params=pltpu.CompilerParams(
            dimension_semantics=("parallel",),
            vmem_limit_bytes=64 * 1024 * 1024),
    )(ct, x, masks, cw, wj, bj, w01, b01, w02, b02,
      w11, b11, w12, b12, wu, bu)

    out = out4.reshape(N, 2, 2, UHalf, H, W)
    out = jnp.transpose(out, (0, 3, 4, 1, 5, 2))
    return out.reshape(N, UHalf, 2 * H, 2 * W)


# speedup vs baseline: 1.5718x; 1.0671x over previous
import jax
import jax.numpy as jnp
import numpy as np
from jax.experimental import pallas as pl
from jax.experimental.pallas import tpu as pltpu


def _sigmoid(v):
    return 1.0 / (1.0 + jnp.exp(-v))


def _flat9(w):
    cout, cin = w.shape[:2]
    return jnp.transpose(w, (0, 2, 3, 1)).reshape(cout, 9 * cin)


def _make_body(ngf, nef, H, W):
    HW = H * W
    C = ngf
    C2 = 2 * ngf
    UC = 2 * (ngf // 2)
    UH = UC // 2

    def col_stack(x, m_first, m_last):
        cin = x.shape[0]
        z1 = jnp.zeros((cin, 1), x.dtype)
        left = jnp.concatenate([z1, x[:, :-1]], axis=1) * m_first
        right = jnp.concatenate([x[:, 1:], z1], axis=1) * m_last
        return jnp.concatenate([left, x, right], axis=0)

    def row_shift(x3, dr):
        zr = jnp.zeros((x3.shape[0], W), x3.dtype)
        if dr == -1:
            return jnp.concatenate([zr, x3[:, :-W]], axis=1)
        return jnp.concatenate([x3[:, W:], zr], axis=1)

    def stack9(x, m_first, m_last):
        x3 = col_stack(x, m_first, m_last)
        return jnp.concatenate(
            [row_shift(x3, -1), x3, row_shift(x3, 1)], axis=0)

    def body(ct_ref, h_ref, mask_ref, cw_ref, wj_ref, bj_ref,
             w01_ref, b01_ref, w02_ref, b02_ref,
             w11_ref, b11_ref, w12_ref, b12_ref,
             wu_ref, bu_ref, o_ref):
        masks = mask_ref[...]
        m_top, m_bot = masks[2:3], masks[3:4]
        mb = masks.astype(jnp.bfloat16)
        m_first, m_last = mb[0:1], mb[1:2]

        x = h_ref[0]

        x9 = stack9(x, m_first, m_last)
        acc = jnp.dot(wj_ref[...], x9,
                      preferred_element_type=jnp.float32)

        vt = jnp.dot(cw_ref[...], ct_ref[0],
                     preferred_element_type=jnp.float32)
        corr = bj_ref[...]
        row_ms = (m_top, None, m_bot)
        col_ms = (m_first, None, m_last)
        for a in range(3):
            s = None
            for b in range(3):
                t = a * 3 + b
                v = vt[t * C2:(t + 1) * C2]
                term = v * col_ms[b] if col_ms[b] is not None else v
                s = term if s is None else s + term
            if row_ms[a] is not None:
                s = s * row_ms[a]
            corr = corr + s
        acc = acc + corr
        y = (acc[:C] * _sigmoid(acc[C:])).astype(jnp.bfloat16)

        def res(yin, w1_ref, b1_ref, w2_ref, b2_ref):
            y9 = stack9(yin, m_first, m_last)
            a1 = jnp.dot(w1_ref[...], y9,
                         preferred_element_type=jnp.float32) + b1_ref[...]
            g = (a1[:C] * _sigmoid(a1[C:])).astype(jnp.bfloat16)
            g9 = stack9(g, m_first, m_last)
            a2 = jnp.dot(w2_ref[...], g9,
                         preferred_element_type=jnp.float32) + b2_ref[...]
            return (a2 + yin.astype(jnp.float32)).astype(jnp.bfloat16)

        z = res(y, w01_ref, b01_ref, w02_ref, b02_ref)
        z = res(z, w11_ref, b11_ref, w12_ref, b12_ref)

        z3 = col_stack(z, m_first, m_last)
        zm = row_shift(z3, -1)
        zp = row_shift(z3, 1)
        for ph, (py, px) in enumerate(((0, 0), (0, 1), (1, 0), (1, 1))):
            za, zb = (zm, z3) if py == 0 else (z3, zp)
            s0 = 0 if px == 0 else C
            xin = jnp.concatenate(
                [za[s0:s0 + 2 * C], zb[s0:s0 + 2 * C]], axis=0)
            aph = jnp.dot(wu_ref[ph], xin,
                          preferred_element_type=jnp.float32) + bu_ref[...]
            o_ref[0, ph] = aph[:UH] * _sigmoid(aph[UH:])

    return body


def kernel(h_code, c_code, joint_w, joint_s, joint_b,
           res0_w1, res0_s1, res0_b1, res0_w2, res0_s2, res0_b2,
           res1_w1, res1_s1, res1_b1, res1_w2, res1_s2, res1_b2,
           up_w, up_s, up_b):
    N, ngf, s, _ = h_code.shape
    nef = c_code.shape[1]
    H = W = s
    HW = H * W
    f32 = jnp.float32
    C, C2 = ngf, 2 * ngf
    UC = 2 * (ngf // 2)

    bf16 = jnp.bfloat16
    x = h_code.astype(bf16).reshape(N, C, HW)
    ct = c_code.astype(f32)[:, :, None]

    wj_f = joint_w.astype(f32) * joint_s.astype(f32)[:, None, None, None]
    wc = wj_f[:, :nef]
    wh = wj_f[:, nef:]
    cw = jnp.transpose(wc, (2, 3, 0, 1)).reshape(9 * C2, nef)
    wj = _flat9(wh).astype(bf16)
    bj = joint_b.astype(f32).reshape(C2, 1)

    def res_prep(w1, s1, b1, w2, s2, b2):
        w1f = w1.astype(f32) * s1.astype(f32)[:, None, None, None]
        w2f = w2.astype(f32) * s2.astype(f32)[:, None, None, None]
        return (_flat9(w1f).astype(bf16), b1.astype(f32).reshape(C2, 1),
                _flat9(w2f).astype(bf16), b2.astype(f32).reshape(C, 1))

    w01, b01, w02, b02 = res_prep(res0_w1, res0_s1, res0_b1,
                                  res0_w2, res0_s2, res0_b2)
    w11, b11, w12, b12 = res_prep(res1_w1, res1_s1, res1_b1,
                                  res1_w2, res1_s2, res1_b2)

    wu_f = up_w.astype(f32) * up_s.astype(f32)[:, None, None, None]
    w9u = jnp.transpose(wu_f, (2, 3, 0, 1)).reshape(9, UC, C)
    row_sets = (((0,), (1, 2)), ((0, 1), (2,)))
    col_sets = (((0,), (1, 2)), ((0, 1), (2,)))
    phases = []
    for py in range(2):
        for px in range(2):
            blocks = []
            for a in range(2):
                for b in range(2):
                    acc = sum(w9u[kh * 3 + kw]
                              for kh in row_sets[py][a]
                              for kw in col_sets[px][b])
                    blocks.append(acc)
            phases.append(jnp.concatenate(blocks, axis=1))
    wu = jnp.stack(phases).astype(bf16)
    bu = up_b.astype(f32).reshape(UC, 1)

    col = np.arange(HW) % W
    row = np.arange(HW) // W
    masks = jnp.asarray(np.stack([
        col != 0, col != W - 1, row != 0, row != H - 1]).astype(np.float32))

    body = _make_body(ngf, nef, H, W)
    UHalf = UC // 2

    out4 = pl.pallas_call(
        body,
        out_shape=jax.ShapeDtypeStruct((N, 4, UHalf, HW), f32),
        grid=(N,),
        in_specs=[
            pl.BlockSpec((1, nef, 1), lambda n: (n, 0, 0)),
            pl.BlockSpec((1, C, HW), lambda n: (n, 0, 0)),
            pl.BlockSpec((4, HW), lambda n: (0, 0)),
            pl.BlockSpec((9 * C2, nef), lambda n: (0, 0)),
            pl.BlockSpec((C2, 9 * C), lambda n: (0, 0)),
            pl.BlockSpec((C2, 1), lambda n: (0, 0)),
            pl.BlockSpec((C2, 9 * C), lambda n: (0, 0)),
            pl.BlockSpec((C2, 1), lambda n: (0, 0)),
            pl.BlockSpec((C, 9 * C), lambda n: (0, 0)),
            pl.BlockSpec((C, 1), lambda n: (0, 0)),
            pl.BlockSpec((C2, 9 * C), lambda n: (0, 0)),
            pl.BlockSpec((C2, 1), lambda n: (0, 0)),
            pl.BlockSpec((C, 9 * C), lambda n: (0, 0)),
            pl.BlockSpec((C, 1), lambda n: (0, 0)),
            pl.BlockSpec((4, UC, 4 * C), lambda n: (0, 0, 0)),
            pl.BlockSpec((UC, 1), lambda n: (0, 0)),
        ],
        out_specs=pl.BlockSpec((1, 4, UHalf, HW), lambda n: (n, 0, 0, 0)),
        compiler_params=pltpu.CompilerParams(
            dimension_semantics=("parallel",),
            vmem_limit_bytes=64 * 1024 * 1024),
    )(ct, x, masks, cw, wj, bj, w01, b01, w02, b02,
      w11, b11, w12, b12, wu, bu)

    out = out4.reshape(N, 2, 2, UHalf, H, W)
    out = jnp.transpose(out, (0, 3, 4, 1, 5, 2))
    return out.reshape(N, UHalf, 2 * H, 2 * W)


# speedup vs baseline: 1.6423x; 1.0449x over previous
import jax
import jax.numpy as jnp
import numpy as np
from jax.experimental import pallas as pl
from jax.experimental.pallas import tpu as pltpu


def _sigmoid(v):
    return 1.0 / (1.0 + jnp.exp(-v))


def _flat9(w):
    cout, cin = w.shape[:2]
    return jnp.transpose(w, (0, 2, 3, 1)).reshape(cout, 9 * cin)


def _make_body(ngf, nef, H, W):
    HW = H * W
    C = ngf
    C2 = 2 * ngf
    UC = 2 * (ngf // 2)
    UH = UC // 2

    def col_stack(x, m_first, m_last):
        cin = x.shape[0]
        z1 = jnp.zeros((cin, 1), x.dtype)
        left = jnp.concatenate([z1, x[:, :-1]], axis=1) * m_first
        right = jnp.concatenate([x[:, 1:], z1], axis=1) * m_last
        return jnp.concatenate([left, x, right], axis=0)

    def row_shift(x3, dr):
        zr = jnp.zeros((x3.shape[0], W), x3.dtype)
        if dr == -1:
            return jnp.concatenate([zr, x3[:, :-W]], axis=1)
        return jnp.concatenate([x3[:, W:], zr], axis=1)

    def stack9(x, m_first, m_last):
        x3 = col_stack(x, m_first, m_last)
        return jnp.concatenate(
            [row_shift(x3, -1), x3, row_shift(x3, 1)], axis=0)

    def body(ct_ref, h_ref, mask_ref, cw_ref, wj_ref, bj_ref,
             w01_ref, b01_ref, w02_ref, b02_ref,
             w11_ref, b11_ref, w12_ref, b12_ref,
             wu_ref, bu_ref, o_ref):
        masks = mask_ref[...]
        m_top, m_bot = masks[2:3], masks[3:4]
        mb = masks.astype(jnp.bfloat16)
        m_first, m_last = mb[0:1], mb[1:2]

        x = h_ref[0].astype(jnp.bfloat16)

        x9 = stack9(x, m_first, m_last)
        acc = jnp.dot(wj_ref[...], x9,
                      preferred_element_type=jnp.float32)

        vt = jnp.dot(cw_ref[...], ct_ref[0],
                     preferred_element_type=jnp.float32)
        corr = bj_ref[...]
        row_ms = (m_top, None, m_bot)
        col_ms = (m_first, None, m_last)
        for a in range(3):
            s = None
            for b in range(3):
                t = a * 3 + b
                v = vt[t * C2:(t + 1) * C2]
                term = v * col_ms[b] if col_ms[b] is not None else v
                s = term if s is None else s + term
            if row_ms[a] is not None:
                s = s * row_ms[a]
            corr = corr + s
        acc = acc + corr
        y = (acc[:C] * _sigmoid(acc[C:])).astype(jnp.bfloat16)

        def res(yin, w1_ref, b1_ref, w2_ref, b2_ref):
            y9 = stack9(yin, m_first, m_last)
            a1 = jnp.dot(w1_ref[...], y9,
                         preferred_element_type=jnp.float32) + b1_ref[...]
            g = (a1[:C] * _sigmoid(a1[C:])).astype(jnp.bfloat16)
            g9 = stack9(g, m_first, m_last)
            a2 = jnp.dot(w2_ref[...], g9,
                         preferred_element_type=jnp.float32) + b2_ref[...]
            return (a2 + yin.astype(jnp.float32)).astype(jnp.bfloat16)

        z = res(y, w01_ref, b01_ref, w02_ref, b02_ref)
        z = res(z, w11_ref, b11_ref, w12_ref, b12_ref)

        z3 = col_stack(z, m_first, m_last)
        zm = row_shift(z3, -1)
        zp = row_shift(z3, 1)
        for ph, (py, px) in enumerate(((0, 0), (0, 1), (1, 0), (1, 1))):
            za, zb = (zm, z3) if py == 0 else (z3, zp)
            s0 = 0 if px == 0 else C
            xin = jnp.concatenate(
                [za[s0:s0 + 2 * C], zb[s0:s0 + 2 * C]], axis=0)
            aph = jnp.dot(wu_ref[ph], xin,
                          preferred_element_type=jnp.float32) + bu_ref[...]
            o_ref[0, ph] = (aph[:UH] * _sigmoid(aph[UH:])).astype(o_ref.dtype)

    return body


def kernel(h_code, c_code, joint_w, joint_s, joint_b,
           res0_w1, res0_s1, res0_b1, res0_w2, res0_s2, res0_b2,
           res1_w1, res1_s1, res1_b1, res1_w2, res1_s2, res1_b2,
           up_w, up_s, up_b):
    N, ngf, s, _ = h_code.shape
    nef = c_code.shape[1]
    H = W = s
    HW = H * W
    f32 = jnp.float32
    C, C2 = ngf, 2 * ngf
    UC = 2 * (ngf // 2)

    bf16 = jnp.bfloat16
    x = h_code.reshape(N, C, HW)
    ct = c_code.astype(f32)[:, :, None]

    wj_f = joint_w.astype(f32) * joint_s.astype(f32)[:, None, None, None]
    wc = wj_f[:, :nef]
    wh = wj_f[:, nef:]
    cw = jnp.transpose(wc, (2, 3, 0, 1)).reshape(9 * C2, nef)
    wj = _flat9(wh).astype(bf16)
    bj = joint_b.astype(f32).reshape(C2, 1)

    def res_prep(w1, s1, b1, w2, s2, b2):
        w1f = w1.astype(f32) * s1.astype(f32)[:, None, None, None]
        w2f = w2.astype(f32) * s2.astype(f32)[:, None, None, None]
        return (_flat9(w1f).astype(bf16), b1.astype(f32).reshape(C2, 1),
                _flat9(w2f).astype(bf16), b2.astype(f32).reshape(C, 1))

    w01, b01, w02, b02 = res_prep(res0_w1, res0_s1, res0_b1,
                                  res0_w2, res0_s2, res0_b2)
    w11, b11, w12, b12 = res_prep(res1_w1, res1_s1, res1_b1,
                                  res1_w2, res1_s2, res1_b2)

    wu_f = up_w.astype(f32) * up_s.astype(f32)[:, None, None, None]
    w9u = jnp.transpose(wu_f, (2, 3, 0, 1)).reshape(9, UC, C)
    row_sets = (((0,), (1, 2)), ((0, 1), (2,)))
    col_sets = (((0,), (1, 2)), ((0, 1), (2,)))
    phases = []
    for py in range(2):
        for px in range(2):
            blocks = []
            for a in range(2):
                for b in range(2):
                    acc = sum(w9u[kh * 3 + kw]
                              for kh in row_sets[py][a]
                              for kw in col_sets[px][b])
                    blocks.append(acc)
            phases.append(jnp.concatenate(blocks, axis=1))
    wu = jnp.stack(phases).astype(bf16)
    bu = up_b.astype(f32).reshape(UC, 1)

    col = np.arange(HW) % W
    row = np.arange(HW) // W
    masks = jnp.asarray(np.stack([
        col != 0, col != W - 1, row != 0, row != H - 1]).astype(np.float32))

    body = _make_body(ngf, nef, H, W)
    UHalf = UC // 2

    out4 = pl.pallas_call(
        body,
        out_shape=jax.ShapeDtypeStruct((N, 4, UHalf, HW), bf16),
        grid=(N,),
        in_specs=[
            pl.BlockSpec((1, nef, 1), lambda n: (n, 0, 0)),
            pl.BlockSpec((1, C, HW), lambda n: (n, 0, 0)),
            pl.BlockSpec((4, HW), lambda n: (0, 0)),
            pl.BlockSpec((9 * C2, nef), lambda n: (0, 0)),
            pl.BlockSpec((C2, 9 * C), lambda n: (0, 0)),
            pl.BlockSpec((C2, 1), lambda n: (0, 0)),
            pl.BlockSpec((C2, 9 * C), lambda n: (0, 0)),
            pl.BlockSpec((C2, 1), lambda n: (0, 0)),
            pl.BlockSpec((C, 9 * C), lambda n: (0, 0)),
            pl.BlockSpec((C, 1), lambda n: (0, 0)),
            pl.BlockSpec((C2, 9 * C), lambda n: (0, 0)),
            pl.BlockSpec((C2, 1), lambda n: (0, 0)),
            pl.BlockSpec((C, 9 * C), lambda n: (0, 0)),
            pl.BlockSpec((C, 1), lambda n: (0, 0)),
            pl.BlockSpec((4, UC, 4 * C), lambda n: (0, 0, 0)),
            pl.BlockSpec((UC, 1), lambda n: (0, 0)),
        ],
        out_specs=pl.BlockSpec((1, 4, UHalf, HW), lambda n: (n, 0, 0, 0)),
        compiler_params=pltpu.CompilerParams(
            dimension_semantics=("parallel",),
            vmem_limit_bytes=64 * 1024 * 1024),
    )(ct, x, masks, cw, wj, bj, w01, b01, w02, b02,
      w11, b11, w12, b12, wu, bu)

    out = out4.reshape(N, 2, 2, UHalf, H, W)
    out = jnp.transpose(out, (0, 3, 4, 1, 5, 2)).astype(f32)
    return out.reshape(N, UHalf, 2 * H, 2 * W)
